# Initial kernel scaffold; baseline (speedup 1.0000x reference)
import jax
import jax.numpy as jnp
from jax.experimental import pallas as pl


def kernel(x, edge_index, W1, b1, W2, b2, Wc, bc):
    raise NotImplementedError("write your pallas kernel here")


# speedup vs baseline: 26.6379x; 26.6379x over previous
import functools

import jax
import jax.numpy as jnp
from jax import lax
from jax.experimental import pallas as pl
from jax.experimental.pallas import tpu as pltpu
from jax.experimental.pallas import tpu_sc as plsc

N = 10000
E = 320000
D = 128
NCLS = 10
NC, NS = 2, 16
NW = NC * NS
NPAD = 10240
B = 64
CPT = 157
EPT = CPT * B
E_PAD = NW * EPT
RPS_INIT = 632
RPS_OUT = NPAD // NS

_MESH = dict(core_axis_name="c", subcore_axis_name="s",
             num_cores=NC, num_subcores=NS)


def _fill_idx(didx_v, dibuf, i):
    for j in range(B // 16):
        didx_v[pl.ds(j * 16, 16)] = dibuf[pl.ds(i * B + j * 16, 16)]


def _deg_body(dst2_hbm, out_hbm, ones_v, zeros_v, didx_v, dibuf, acc):
    c = lax.axis_index("c")
    s = lax.axis_index("s")
    wid = s * NC + c
    for i in range(B // 16):
        ones_v[pl.ds(i * 16, 16)] = jnp.full((16,), 1.0, jnp.float32)
        zeros_v[pl.ds(i * 16, 16)] = jnp.zeros((16,), jnp.float32)
    for i in range(RPS_OUT // B):
        pltpu.sync_copy(zeros_v, acc.at[pl.ds(s * RPS_OUT + i * B, B)])
    pltpu.sync_copy(dst2_hbm.at[wid], dibuf)
    plsc.subcore_barrier()

    def body(i, carry):
        _fill_idx(didx_v, dibuf, i)
        pltpu.sync_copy(ones_v, acc.at[didx_v], add=True)
        return carry

    lax.fori_loop(0, CPT, body, 0)
    plsc.subcore_barrier()
    pltpu.sync_copy(acc.at[pl.ds(s * RPS_OUT, RPS_OUT)],
                    out_hbm.at[c, pl.ds(s * RPS_OUT, RPS_OUT)])


@functools.cache
def _deg_call():
    return pl.kernel(
        _deg_body,
        out_type=jax.ShapeDtypeStruct((NC, NPAD), jnp.float32),
        mesh=plsc.VectorSubcoreMesh(**_MESH),
        scratch_types=[
            pltpu.VMEM((B,), jnp.float32),
            pltpu.VMEM((B,), jnp.float32),
            pltpu.VMEM((B,), jnp.int32),
            pltpu.VMEM((EPT,), jnp.int32),
            pltpu.VMEM_SHARED((NPAD,), jnp.float32),
        ],
    )


def _agg_body(hp_hbm, src2_hbm, dst2_hbm, out_hbm,
              sibuf, dibuf, didx_v, rows0, rows1, acc, sem0, sem1):
    c = lax.axis_index("c")
    s = lax.axis_index("s")
    wid = s * NC + c
    base = jnp.minimum(s * RPS_INIT, N - RPS_INIT)
    pltpu.sync_copy(hp_hbm.at[pl.ds(base, RPS_INIT)],
                    acc.at[pl.ds(base, RPS_INIT)])
    pltpu.sync_copy(src2_hbm.at[wid], sibuf)
    pltpu.sync_copy(dst2_hbm.at[wid], dibuf)
    plsc.subcore_barrier()

    def gidx(i):
        return sibuf.at[pl.ds(i * B, B)]

    pltpu.async_copy(hp_hbm.at[gidx(0)], rows0, sem0)

    def body(k, carry):
        i0 = 2 * k
        i1 = i0 + 1
        i2 = i0 + 2
        pltpu.async_copy(hp_hbm.at[gidx(i1)], rows1, sem1)
        pltpu.make_async_copy(hp_hbm.at[gidx(i0)], rows0, sem0).wait()
        _fill_idx(didx_v, dibuf, i0)
        pltpu.sync_copy(rows0, acc.at[didx_v], add=True)
        pltpu.async_copy(hp_hbm.at[gidx(i2)], rows0, sem0)
        pltpu.make_async_copy(hp_hbm.at[gidx(i1)], rows1, sem1).wait()
        _fill_idx(didx_v, dibuf, i1)
        pltpu.sync_copy(rows1, acc.at[didx_v], add=True)
        return carry

    lax.fori_loop(0, (CPT - 1) // 2, body, 0)
    pltpu.make_async_copy(hp_hbm.at[gidx(CPT - 1)], rows0, sem0).wait()
    _fill_idx(didx_v, dibuf, CPT - 1)
    pltpu.sync_copy(rows0, acc.at[didx_v], add=True)
    plsc.subcore_barrier()
    pltpu.sync_copy(acc.at[pl.ds(s * RPS_OUT, RPS_OUT)],
                    out_hbm.at[c, pl.ds(s * RPS_OUT, RPS_OUT)])


@functools.cache
def _agg_call():
    return pl.kernel(
        _agg_body,
        out_type=jax.ShapeDtypeStruct((NC, NPAD, D), jnp.float32),
        mesh=plsc.VectorSubcoreMesh(**_MESH),
        scratch_types=[
            pltpu.VMEM((EPT,), jnp.int32),
            pltpu.VMEM((EPT,), jnp.int32),
            pltpu.VMEM((B,), jnp.int32),
            pltpu.VMEM((B, D), jnp.float32),
            pltpu.VMEM((B, D), jnp.float32),
            pltpu.VMEM_SHARED((NPAD, D), jnp.float32),
            pltpu.SemaphoreType.DMA,
            pltpu.SemaphoreType.DMA,
        ],
    )


RB = 1000


def _tc1_body(x_ref, w_ref, d0_ref, d1_ref, o_ref):
    dis = lax.rsqrt(d0_ref[...] + d1_ref[...] + 1.0)
    h = jnp.dot(x_ref[...], w_ref[...], preferred_element_type=jnp.float32)
    o_ref[...] = h * dis


def _tc2_body(a0_ref, a1_ref, hp_ref, d0_ref, d1_ref, b_ref, w_ref, o_ref):
    dis = lax.rsqrt(d0_ref[...] + d1_ref[...] + 1.0)
    tot = a0_ref[...] + a1_ref[...] - hp_ref[...]
    h1 = jnp.maximum(tot * dis + b_ref[...], 0.0)
    o_ref[...] = jnp.dot(h1, w_ref[...], preferred_element_type=jnp.float32) * dis


def _tc3_body(a0_ref, a1_ref, hp_ref, d0_ref, d1_ref, b_ref, w_ref, bc_ref,
              o_ref):
    dis = lax.rsqrt(d0_ref[...] + d1_ref[...] + 1.0)
    tot = a0_ref[...] + a1_ref[...] - hp_ref[...]
    h2 = jnp.maximum(tot * dis + b_ref[...], 0.0)
    o_ref[...] = (jnp.dot(h2, w_ref[...], preferred_element_type=jnp.float32)
                  + bc_ref[...])


def _row_spec(width):
    return pl.BlockSpec((RB, width), lambda i: (i, 0))


def _full_spec(shape):
    return pl.BlockSpec(shape, lambda i: tuple(0 for _ in shape))


_tc1 = pl.pallas_call(
    _tc1_body,
    grid=(N // RB,),
    in_specs=[_row_spec(D), _full_spec((D, D)), _row_spec(1), _row_spec(1)],
    out_specs=_row_spec(D),
    out_shape=jax.ShapeDtypeStruct((N, D), jnp.float32),
)

_tc2 = pl.pallas_call(
    _tc2_body,
    grid=(N // RB,),
    in_specs=[_row_spec(D), _row_spec(D), _row_spec(D), _row_spec(1),
              _row_spec(1), _full_spec((1, D)), _full_spec((D, D))],
    out_specs=_row_spec(D),
    out_shape=jax.ShapeDtypeStruct((N, D), jnp.float32),
)

_tc3 = pl.pallas_call(
    _tc3_body,
    grid=(N // RB,),
    in_specs=[_row_spec(D), _row_spec(D), _row_spec(D), _row_spec(1),
              _row_spec(1), _full_spec((1, D)), _full_spec((D, NCLS)),
              _full_spec((1, NCLS))],
    out_specs=_row_spec(NCLS),
    out_shape=jax.ShapeDtypeStruct((N, NCLS), jnp.float32),
)


def kernel(x, edge_index, W1, b1, W2, b2, Wc, bc):
    ei = edge_index.astype(jnp.int32)
    npad = jnp.arange(E_PAD - E, dtype=jnp.int32)
    src2 = jnp.concatenate([ei[0], (npad * 997) % N]).reshape(NW, EPT)
    dst2 = jnp.concatenate([ei[1], N + npad % (NPAD - N)]).reshape(NW, EPT)

    deg_parts = _deg_call()(dst2)
    d0 = deg_parts[0].reshape(NPAD, 1)
    d1 = deg_parts[1].reshape(NPAD, 1)

    h1p = _tc1(x, W1, d0, d1)
    p1 = _agg_call()(h1p, src2, dst2)
    h2p = _tc2(p1[0], p1[1], h1p, d0, d1,
               b1.reshape(1, D), W2)
    p2 = _agg_call()(h2p, src2, dst2)
    out = _tc3(p2[0], p2[1], h2p, d0, d1,
               b2.reshape(1, D), Wc, bc.reshape(1, NCLS))
    return out
